# baseline (speedup 1.0000x reference)
import functools

import jax
import jax.numpy as jnp
from jax import lax
from jax.experimental import pallas as pl
from jax.experimental.pallas import tpu as pltpu
from jax.experimental.pallas import tpu_sc as plsc

N = 10000
E = 320000
D = 128

NC = 2
NS = 16
NW = NC * NS
EPW = E // NW
CH = 80
NCHUNK = 128
EPWP = NCHUNK * CH
GPC = 8
NG = NCHUNK // GPC
NPAD = 10240
RCH = NPAD // (NS * CH)

_mesh = plsc.VectorSubcoreMesh(
    core_axis_name="c", subcore_axis_name="s", num_cores=NC, num_subcores=NS
)


_AGRP = 8
_ANG = NCHUNK // _AGRP


@functools.partial(
    pl.kernel,
    out_type=jax.ShapeDtypeStruct((NC, NPAD, 16), jnp.float32),
    mesh=_mesh,
    scratch_types=[
        pltpu.VMEM((NCHUNK, CH), jnp.int32),
        pltpu.VMEM((CH, 16), jnp.float32),
        pltpu.VMEM((CH, 16), jnp.float32),
        pltpu.VMEM_SHARED((NPAD, 16), jnp.float32),
        pltpu.SemaphoreType.DMA,
        pltpu.SemaphoreType.DMA,
        pltpu.SemaphoreType.DMA,
    ],
)
def _deg_kernel(dst_hbm, out_hbm, didx_v, ones_v, zeros_v, hist_sh, semi, sema, semw):
    c = lax.axis_index("c")
    s = lax.axis_index("s")
    wid = s * NC + c

    pltpu.async_copy(dst_hbm.at[wid], didx_v, semi)

    @pl.loop(0, CH)
    def _fill(r):
        ones_v[r] = jnp.ones((16,), jnp.float32)
        zeros_v[r] = jnp.zeros((16,), jnp.float32)

    @pl.loop(0, RCH)
    def _zero(k):
        pltpu.sync_copy(zeros_v, hist_sh.at[pl.ds((s * RCH + k) * CH, CH)])

    pltpu.make_async_copy(dst_hbm.at[wid], didx_v, semi).wait()
    plsc.subcore_barrier()

    def _add_start(k):
        pltpu.async_copy(ones_v, hist_sh.at[didx_v.at[k]], sema, add=True)

    def _add_wait():
        pltpu.make_async_copy(ones_v, hist_sh.at[didx_v.at[0]], sema).wait()

    for j in range(_AGRP):
        _add_start(j)

    @pl.loop(0, _ANG - 1)
    def _accum(i):
        for j in range(_AGRP):
            _add_start((i + 1) * _AGRP + j)
        for j in range(_AGRP):
            _add_wait()

    for j in range(_AGRP):
        _add_wait()

    plsc.subcore_barrier()

    @pl.loop(0, RCH)
    def _writeback(k):
        r0 = (s * RCH + k) * CH
        pltpu.sync_copy(hist_sh.at[pl.ds(r0, CH)], zeros_v)
        pltpu.sync_copy(zeros_v, out_hbm.at[c, pl.ds(r0, CH)])


_RB = 2000


def _lin_body(x_ref, m_ref, h0_ref, h1_ref, w_ref, g_ref, s_ref):
    deg = 1.0 + h0_ref[...] + h1_ref[...]
    sv = m_ref[...] * lax.rsqrt(deg)
    s_ref[...] = sv
    g_ref[...] = jnp.dot(
        x_ref[...] * sv, w_ref[...], preferred_element_type=jnp.float32
    )


_linear = pl.pallas_call(
    _lin_body,
    grid=(N // _RB,),
    in_specs=[
        pl.BlockSpec((_RB, D), lambda i: (i, 0)),
        pl.BlockSpec((_RB, 1), lambda i: (i, 0)),
        pl.BlockSpec((_RB, 1), lambda i: (i, 0)),
        pl.BlockSpec((_RB, 1), lambda i: (i, 0)),
        pl.BlockSpec((D, D), lambda i: (0, 0)),
    ],
    out_specs=[
        pl.BlockSpec((_RB, D), lambda i: (i, 0)),
        pl.BlockSpec((_RB, 1), lambda i: (i, 0)),
    ],
    out_shape=[
        jax.ShapeDtypeStruct((N, D), jnp.float32),
        jax.ShapeDtypeStruct((N, 1), jnp.float32),
    ],
)


@functools.partial(
    pl.kernel,
    out_type=jax.ShapeDtypeStruct((NC, NPAD, D), jnp.float32),
    mesh=_mesh,
    scratch_types=[
        pltpu.VMEM((2, GPC, CH), jnp.int32),
        pltpu.VMEM((2, GPC, CH), jnp.int32),
        pltpu.VMEM((2, CH, D), jnp.float32),
        pltpu.VMEM_SHARED((NPAD, D), jnp.float32),
        pltpu.SemaphoreType.DMA((2,)),
        pltpu.SemaphoreType.DMA((2,)),
        pltpu.SemaphoreType.DMA,
    ],
)
def _edge_kernel(
    src_hbm, dst_hbm, g_hbm, out_hbm,
    sidx_v, didx_v, rows_v, acc_sh, semr, semg, semw,
):
    c = lax.axis_index("c")
    s = lax.axis_index("s")
    wid = s * NC + c

    def _refill_start(grp, ib):
        pltpu.async_copy(
            src_hbm.at[wid, pl.ds(grp * GPC, GPC)], sidx_v.at[ib], semr.at[ib]
        )
        pltpu.async_copy(
            dst_hbm.at[wid, pl.ds(grp * GPC, GPC)], didx_v.at[ib], semr.at[ib]
        )

    def _refill_wait(ib):
        pltpu.make_async_copy(
            src_hbm.at[wid, pl.ds(0, GPC)], sidx_v.at[ib], semr.at[ib]
        ).wait()
        pltpu.make_async_copy(
            dst_hbm.at[wid, pl.ds(0, GPC)], didx_v.at[ib], semr.at[ib]
        ).wait()

    def _gather_start(ib, j, sl):
        pltpu.async_copy(g_hbm.at[sidx_v.at[ib, j]], rows_v.at[sl], semg.at[sl])

    def _gather_wait(sl):
        pltpu.make_async_copy(
            g_hbm.at[sidx_v.at[0, 0]], rows_v.at[sl], semg.at[sl]
        ).wait()

    _refill_start(0, 0)
    _refill_start(1, 1)

    @pl.loop(0, CH)
    def _zero_rows(r):
        for j in range(D // 16):
            rows_v[0, r, pl.ds(j * 16, 16)] = jnp.zeros((16,), jnp.float32)

    @pl.loop(0, RCH)
    def _zero_acc(k):
        pltpu.sync_copy(rows_v.at[0], acc_sh.at[pl.ds((s * RCH + k) * CH, CH)])

    plsc.subcore_barrier()

    _refill_wait(0)
    _gather_start(0, 0, 0)

    @pl.loop(0, NG)
    def _grp(grp):
        ib = lax.rem(grp, 2)
        for j in range(GPC):
            sl = j % 2
            if j < GPC - 1:
                _gather_start(ib, j + 1, 1 - sl)
            else:
                @pl.when(grp < NG - 1)
                def _next_group():
                    _refill_wait(1 - ib)
                    _gather_start(1 - ib, 0, 1 - sl)

            _gather_wait(sl)
            pltpu.sync_copy(rows_v.at[sl], acc_sh.at[didx_v.at[ib, j]], add=True)

        @pl.when(grp < NG - 2)
        def _prefetch_idx():
            _refill_start(grp + 2, ib)

    plsc.subcore_barrier()

    @pl.loop(0, RCH)
    def _writeback(k):
        r0 = (s * RCH + k) * CH
        pltpu.sync_copy(acc_sh.at[pl.ds(r0, CH)], rows_v.at[0])
        pltpu.sync_copy(rows_v.at[0], out_hbm.at[c, pl.ds(r0, CH)])


def _fin_body(p0_ref, p1_ref, g_ref, s_ref, m_ref, b_ref, o_ref):
    acc = p0_ref[...] + p1_ref[...] + g_ref[...]
    o_ref[...] = s_ref[...] * acc + m_ref[...] * b_ref[...]


_final = pl.pallas_call(
    _fin_body,
    grid=(N // _RB,),
    in_specs=[
        pl.BlockSpec((_RB, D), lambda i: (i, 0)),
        pl.BlockSpec((_RB, D), lambda i: (i, 0)),
        pl.BlockSpec((_RB, D), lambda i: (i, 0)),
        pl.BlockSpec((_RB, 1), lambda i: (i, 0)),
        pl.BlockSpec((_RB, 1), lambda i: (i, 0)),
        pl.BlockSpec((1, D), lambda i: (0, 0)),
    ],
    out_specs=pl.BlockSpec((_RB, D), lambda i: (i, 0)),
    out_shape=jax.ShapeDtypeStruct((N, D), jnp.float32),
)


def kernel(x, edge_index, mask, W, b):
    pad = jnp.full((NW, EPWP - EPW), N, jnp.int32)
    src = jnp.concatenate(
        [edge_index[0].reshape(NW, EPW), pad], axis=1
    ).reshape(NW, NCHUNK, CH)
    dst = jnp.concatenate(
        [edge_index[1].reshape(NW, EPW), pad], axis=1
    ).reshape(NW, NCHUNK, CH)
    mask_f = mask.astype(jnp.float32).reshape(N, 1)
    hist = _deg_kernel(dst)
    h0 = hist[0, :N, 0:1]
    h1 = hist[1, :N, 0:1]
    g, sv = _linear(x, mask_f, h0, h1, W)
    g_pad = jnp.concatenate([g, jnp.zeros((8, D), jnp.float32)], axis=0)
    p = _edge_kernel(src, dst, g_pad)
    return _final(p[0, :N], p[1, :N], g, sv, mask_f, b.reshape(1, D))


# speedup vs baseline: 1.0284x; 1.0284x over previous
import functools

import jax
import jax.numpy as jnp
from jax import lax
from jax.experimental import pallas as pl
from jax.experimental.pallas import tpu as pltpu
from jax.experimental.pallas import tpu_sc as plsc

N = 10000
E = 320000
D = 128

NC = 2
NS = 16
NW = NC * NS
EPW = E // NW
CH = 128
NCHUNK = 80
EPWP = NCHUNK * CH
GPC = 8
NG = NCHUNK // GPC
NPAD = 10240
RCH = NPAD // (NS * CH)

_mesh = plsc.VectorSubcoreMesh(
    core_axis_name="c", subcore_axis_name="s", num_cores=NC, num_subcores=NS
)


_AGRP = 8
_ANG = NCHUNK // _AGRP


@functools.partial(
    pl.kernel,
    out_type=jax.ShapeDtypeStruct((NC, NPAD, 16), jnp.float32),
    mesh=_mesh,
    scratch_types=[
        pltpu.VMEM((NCHUNK, CH), jnp.int32),
        pltpu.VMEM((CH, 16), jnp.float32),
        pltpu.VMEM((CH, 16), jnp.float32),
        pltpu.VMEM_SHARED((NPAD, 16), jnp.float32),
        pltpu.SemaphoreType.DMA,
        pltpu.SemaphoreType.DMA,
        pltpu.SemaphoreType.DMA,
    ],
)
def _deg_kernel(dst_hbm, out_hbm, didx_v, ones_v, zeros_v, hist_sh, semi, sema, semw):
    c = lax.axis_index("c")
    s = lax.axis_index("s")
    wid = s * NC + c

    pltpu.async_copy(dst_hbm.at[wid], didx_v, semi)

    @pl.loop(0, CH)
    def _fill(r):
        ones_v[r] = jnp.ones((16,), jnp.float32)
        zeros_v[r] = jnp.zeros((16,), jnp.float32)

    @pl.loop(0, RCH)
    def _zero(k):
        pltpu.sync_copy(zeros_v, hist_sh.at[pl.ds((s * RCH + k) * CH, CH)])

    pltpu.make_async_copy(dst_hbm.at[wid], didx_v, semi).wait()
    plsc.subcore_barrier()

    def _add_start(k):
        pltpu.async_copy(ones_v, hist_sh.at[didx_v.at[k]], sema, add=True)

    def _add_wait():
        pltpu.make_async_copy(ones_v, hist_sh.at[didx_v.at[0]], sema).wait()

    for j in range(_AGRP):
        _add_start(j)

    @pl.loop(0, _ANG - 1)
    def _accum(i):
        for j in range(_AGRP):
            _add_start((i + 1) * _AGRP + j)
        for j in range(_AGRP):
            _add_wait()

    for j in range(_AGRP):
        _add_wait()

    plsc.subcore_barrier()

    @pl.loop(0, RCH)
    def _writeback(k):
        r0 = (s * RCH + k) * CH
        pltpu.sync_copy(hist_sh.at[pl.ds(r0, CH)], zeros_v)
        pltpu.sync_copy(zeros_v, out_hbm.at[c, pl.ds(r0, CH)])


_RB = 2000


def _lin_body(x_ref, m_ref, h0_ref, h1_ref, w_ref, g_ref, s_ref):
    deg = 1.0 + h0_ref[...] + h1_ref[...]
    sv = m_ref[...] * lax.rsqrt(deg)
    s_ref[...] = sv
    g_ref[...] = jnp.dot(
        x_ref[...] * sv, w_ref[...], preferred_element_type=jnp.float32
    )


_linear = pl.pallas_call(
    _lin_body,
    grid=(N // _RB,),
    in_specs=[
        pl.BlockSpec((_RB, D), lambda i: (i, 0)),
        pl.BlockSpec((_RB, 1), lambda i: (i, 0)),
        pl.BlockSpec((_RB, 1), lambda i: (i, 0)),
        pl.BlockSpec((_RB, 1), lambda i: (i, 0)),
        pl.BlockSpec((D, D), lambda i: (0, 0)),
    ],
    out_specs=[
        pl.BlockSpec((_RB, D), lambda i: (i, 0)),
        pl.BlockSpec((_RB, 1), lambda i: (i, 0)),
    ],
    out_shape=[
        jax.ShapeDtypeStruct((N, D), jnp.float32),
        jax.ShapeDtypeStruct((N, 1), jnp.float32),
    ],
)


@functools.partial(
    pl.kernel,
    out_type=jax.ShapeDtypeStruct((NC, NPAD, D), jnp.float32),
    mesh=_mesh,
    scratch_types=[
        pltpu.VMEM((GPC, CH), jnp.int32),
        pltpu.VMEM((GPC, CH), jnp.int32),
        pltpu.VMEM((GPC, CH), jnp.int32),
        pltpu.VMEM((GPC, CH), jnp.int32),
        pltpu.VMEM((CH, D), jnp.float32),
        pltpu.VMEM((CH, D), jnp.float32),
        pltpu.VMEM_SHARED((NPAD, D), jnp.float32),
        pltpu.SemaphoreType.DMA,
        pltpu.SemaphoreType.DMA,
        pltpu.SemaphoreType.DMA,
        pltpu.SemaphoreType.DMA,
        pltpu.SemaphoreType.DMA,
        pltpu.SemaphoreType.DMA,
    ],
)
def _edge_kernel(
    src_hbm, dst_hbm, g_hbm, out_hbm,
    sia, sib, dia, dib, rows0, rows1, acc_sh,
    semra, semrb, semg0, semg1, sema0, sema1,
):
    c = lax.axis_index("c")
    s = lax.axis_index("s")
    wid = s * NC + c

    sbuf = (sia, sib)
    dbuf = (dia, dib)
    rows = (rows0, rows1)
    semr = (semra, semrb)
    semg = (semg0, semg1)
    sema = (sema0, sema1)

    def _refill_start(grp, b):
        pltpu.async_copy(
            src_hbm.at[wid, pl.ds(grp * GPC, GPC)], sbuf[b], semr[b]
        )
        pltpu.async_copy(
            dst_hbm.at[wid, pl.ds(grp * GPC, GPC)], dbuf[b], semr[b]
        )

    def _refill_wait(b):
        pltpu.make_async_copy(
            src_hbm.at[wid, pl.ds(0, GPC)], sbuf[b], semr[b]
        ).wait()
        pltpu.make_async_copy(
            dst_hbm.at[wid, pl.ds(0, GPC)], dbuf[b], semr[b]
        ).wait()

    def _gather_start(b, jj, sl):
        pltpu.async_copy(g_hbm.at[sbuf[b].at[jj]], rows[sl], semg[sl])

    def _gather_wait(sl):
        pltpu.make_async_copy(g_hbm.at[sia.at[0]], rows[sl], semg[sl]).wait()

    def _add_start(b, jj, sl):
        pltpu.async_copy(rows[sl], acc_sh.at[dbuf[b].at[jj]], sema[sl], add=True)

    def _add_wait(sl):
        pltpu.make_async_copy(rows[sl], acc_sh.at[dia.at[0]], sema[sl]).wait()

    _refill_start(0, 0)
    _refill_start(1, 1)

    @pl.loop(0, CH)
    def _zero_rows(r):
        for j in range(D // 16):
            rows0[r, pl.ds(j * 16, 16)] = jnp.zeros((16,), jnp.float32)

    @pl.loop(0, RCH)
    def _zero_acc(k):
        pltpu.sync_copy(rows0, acc_sh.at[pl.ds((s * RCH + k) * CH, CH)])

    plsc.subcore_barrier()

    _refill_wait(0)
    _gather_start(0, 0, 0)

    @pl.loop(0, NG // 2)
    def _pair(p):
        for j in range(2 * GPC):
            b, jj = (0, j) if j < GPC else (1, j - GPC)
            sl = j % 2
            ns = 1 - sl

            if j == 0:
                @pl.when(p > 0)
                def _drain0():
                    _add_wait(ns)
            else:
                _add_wait(ns)

            if j == 1:
                @pl.when(p > 0)
                def _refill_b():
                    _refill_start(2 * p + 1, 1)
            if j == GPC:
                @pl.when(p < NG // 2 - 1)
                def _refill_a():
                    _refill_start(2 * p + 2, 0)

            if j == GPC - 1:
                _refill_wait(1)

            if j < 2 * GPC - 1:
                nb, njj = (0, j + 1) if j + 1 < GPC else (1, j + 1 - GPC)
                _gather_start(nb, njj, ns)
            else:
                @pl.when(p < NG // 2 - 1)
                def _next_pair():
                    _refill_wait(0)
                    _gather_start(0, 0, ns)

            _gather_wait(sl)
            _add_start(b, jj, sl)

    _add_wait(1)

    plsc.subcore_barrier()

    @pl.loop(0, RCH)
    def _writeback(k):
        r0 = (s * RCH + k) * CH
        pltpu.sync_copy(acc_sh.at[pl.ds(r0, CH)], rows0)
        pltpu.sync_copy(rows0, out_hbm.at[c, pl.ds(r0, CH)])


def _fin_body(p0_ref, p1_ref, g_ref, s_ref, m_ref, b_ref, o_ref):
    acc = p0_ref[...] + p1_ref[...] + g_ref[...]
    o_ref[...] = s_ref[...] * acc + m_ref[...] * b_ref[...]


_final = pl.pallas_call(
    _fin_body,
    grid=(N // _RB,),
    in_specs=[
        pl.BlockSpec((_RB, D), lambda i: (i, 0)),
        pl.BlockSpec((_RB, D), lambda i: (i, 0)),
        pl.BlockSpec((_RB, D), lambda i: (i, 0)),
        pl.BlockSpec((_RB, 1), lambda i: (i, 0)),
        pl.BlockSpec((_RB, 1), lambda i: (i, 0)),
        pl.BlockSpec((1, D), lambda i: (0, 0)),
    ],
    out_specs=pl.BlockSpec((_RB, D), lambda i: (i, 0)),
    out_shape=jax.ShapeDtypeStruct((N, D), jnp.float32),
)


def kernel(x, edge_index, mask, W, b):
    pad = jnp.full((NW, EPWP - EPW), N, jnp.int32)
    src = jnp.concatenate(
        [edge_index[0].reshape(NW, EPW), pad], axis=1
    ).reshape(NW, NCHUNK, CH)
    dst = jnp.concatenate(
        [edge_index[1].reshape(NW, EPW), pad], axis=1
    ).reshape(NW, NCHUNK, CH)
    mask_f = mask.astype(jnp.float32).reshape(N, 1)
    hist = _deg_kernel(dst)
    h0 = hist[0, :N, 0:1]
    h1 = hist[1, :N, 0:1]
    g, sv = _linear(x, mask_f, h0, h1, W)
    g_pad = jnp.concatenate([g, jnp.zeros((8, D), jnp.float32)], axis=0)
    p = _edge_kernel(src, dst, g_pad)
    return _final(p[0, :N], p[1, :N], g, sv, mask_f, b.reshape(1, D))
